# initial kernel scaffold (unmeasured)
import jax
import jax.numpy as jnp
from jax import lax
from jax.experimental import pallas as pl
from jax.experimental.pallas import tpu as pltpu


def kernel(
    u,
):
    def body(*refs):
        pass

    out_shape = jax.ShapeDtypeStruct(..., jnp.float32)
    return pl.pallas_call(body, out_shape=out_shape)(...)


# baseline (device time: 13746 ns/iter reference)
import jax
import jax.numpy as jnp
from jax import lax
from jax.experimental import pallas as pl
from jax.experimental.pallas import tpu as pltpu

NX, NY, NZ = 2, 2, 4

_DEVICE_ID_TYPE = getattr(pl, "DeviceIdType", None) or pltpu.DeviceIdType


def kernel(u):
    sx, sy, sz = u.shape

    def body(x_ref, out_ref, send_buf, recv_buf, send_sems, recv_sems):
        my_x = lax.axis_index("x")
        my_y = lax.axis_index("y")
        my_z = lax.axis_index("z")

        uv = x_ref[...]

        send_buf[0, :, :] = uv[0, :, :]
        send_buf[1, :, :] = uv[sx - 1, :, :]
        send_buf[2, :, :] = uv[:, 0, :]
        send_buf[3, :, :] = uv[:, sy - 1, :]
        send_buf[4, :, :] = uv[:, :, 0]
        send_buf[5, :, :] = uv[:, :, sz - 1]

        preds = [
            my_x > 0, my_x < NX - 1,
            my_y > 0, my_y < NY - 1,
            my_z > 0, my_z < NZ - 1,
        ]
        nbrs = [
            (my_x - 1, my_y, my_z), (my_x + 1, my_y, my_z),
            (my_x, my_y - 1, my_z), (my_x, my_y + 1, my_z),
            (my_x, my_y, my_z - 1), (my_x, my_y, my_z + 1),
        ]
        opp = [1, 0, 3, 2, 5, 4]

        def start_send(d):
            @pl.when(preds[d])
            def _():
                rdma = pltpu.make_async_remote_copy(
                    src_ref=send_buf.at[d],
                    dst_ref=recv_buf.at[opp[d]],
                    send_sem=send_sems.at[d],
                    recv_sem=recv_sems.at[opp[d]],
                    device_id=nbrs[d],
                    device_id_type=_DEVICE_ID_TYPE.MESH,
                )
                rdma.start()

        def wait_exchange(d):
            @pl.when(preds[d])
            def _():
                w = pltpu.make_async_remote_copy(
                    src_ref=send_buf.at[d],
                    dst_ref=recv_buf.at[d],
                    send_sem=send_sems.at[d],
                    recv_sem=recv_sems.at[d],
                    device_id=nbrs[d],
                    device_id_type=_DEVICE_ID_TYPE.MESH,
                )
                w.wait_send()
                w.wait_recv()

        for d in range(6):
            start_send(d)
        for d in range(6):
            wait_exchange(d)

        xlo = recv_buf[0, :, :]
        xhi = recv_buf[1, :, :]
        ylo = recv_buf[2, :, :]
        yhi = recv_buf[3, :, :]
        zlo = recv_buf[4, :, :]
        zhi = recv_buf[5, :, :]

        xm = jnp.concatenate([xlo[None, :, :], uv[:-1, :, :]], axis=0)
        xp = jnp.concatenate([uv[1:, :, :], xhi[None, :, :]], axis=0)
        ym = jnp.concatenate([ylo[:, None, :], uv[:, :-1, :]], axis=1)
        yp = jnp.concatenate([uv[:, 1:, :], yhi[:, None, :]], axis=1)
        zm = jnp.concatenate([zlo[:, :, None], uv[:, :, :-1]], axis=2)
        zp = jnp.concatenate([uv[:, :, 1:], zhi[:, :, None]], axis=2)

        v = xm + xp + ym + yp + zm + zp - 6.0 * uv

        ii = lax.broadcasted_iota(jnp.int32, (sx, sy, sz), 0)
        jj = lax.broadcasted_iota(jnp.int32, (sx, sy, sz), 1)
        kk = lax.broadcasted_iota(jnp.int32, (sx, sy, sz), 2)
        gi = my_x * sx + ii
        gj = my_y * sy + jj
        gk = my_z * sz + kk
        interior = (
            (gi > 0) & (gi < NX * sx - 1)
            & (gj > 0) & (gj < NY * sy - 1)
            & (gk > 0) & (gk < NZ * sz - 1)
        )
        out_ref[...] = jnp.where(interior, v, 0.0)

    return pl.pallas_call(
        body,
        out_shape=jax.ShapeDtypeStruct((sx, sy, sz), jnp.float32),
        in_specs=[pl.BlockSpec(memory_space=pltpu.VMEM)],
        out_specs=pl.BlockSpec(memory_space=pltpu.VMEM),
        scratch_shapes=[
            pltpu.VMEM((6, sy, sz), jnp.float32),
            pltpu.VMEM((6, sy, sz), jnp.float32),
            pltpu.SemaphoreType.DMA((6,)),
            pltpu.SemaphoreType.DMA((6,)),
        ],
    )(u)


# device time: 7613 ns/iter; 1.8056x vs baseline; 1.8056x over previous
import jax
import jax.numpy as jnp
from jax import lax
from jax.experimental import pallas as pl
from jax.experimental.pallas import tpu as pltpu

NX, NY, NZ = 2, 2, 4

_DEVICE_ID_TYPE = getattr(pl, "DeviceIdType", None) or pltpu.DeviceIdType


def kernel(u):
    sx, sy, sz = u.shape

    def body(x_ref, out_ref, send_buf, recv_buf, send_sems, recv_sems):
        my_x = lax.axis_index("x")
        my_y = lax.axis_index("y")
        my_z = lax.axis_index("z")

        uv = x_ref[...]

        send_buf[0, :, :] = uv[0, :, :]
        send_buf[1, :, :] = uv[sx - 1, :, :]
        send_buf[2, :, :] = uv[:, 0, :]
        send_buf[3, :, :] = uv[:, sy - 1, :]
        send_buf[4, :, :] = uv[:, :, 0]
        send_buf[5, :, :] = uv[:, :, sz - 1]

        preds = [
            my_x > 0, my_x < NX - 1,
            my_y > 0, my_y < NY - 1,
            my_z > 0, my_z < NZ - 1,
        ]
        nbrs = [
            (my_x - 1, my_y, my_z), (my_x + 1, my_y, my_z),
            (my_x, my_y - 1, my_z), (my_x, my_y + 1, my_z),
            (my_x, my_y, my_z - 1), (my_x, my_y, my_z + 1),
        ]
        opp = [1, 0, 3, 2, 5, 4]

        barrier_sem = pltpu.get_barrier_semaphore()
        for d in range(6):
            @pl.when(preds[d])
            def _(d=d):
                pl.semaphore_signal(
                    barrier_sem, inc=1,
                    device_id=nbrs[d],
                    device_id_type=_DEVICE_ID_TYPE.MESH,
                )

            @pl.when(jnp.logical_not(preds[d]))
            def _():
                pl.semaphore_signal(barrier_sem, inc=1)
        pl.semaphore_wait(barrier_sem, 6)

        def start_send(d):
            @pl.when(preds[d])
            def _():
                rdma = pltpu.make_async_remote_copy(
                    src_ref=send_buf.at[d],
                    dst_ref=recv_buf.at[opp[d]],
                    send_sem=send_sems.at[d],
                    recv_sem=recv_sems.at[opp[d]],
                    device_id=nbrs[d],
                    device_id_type=_DEVICE_ID_TYPE.MESH,
                )
                rdma.start()

        for d in range(6):
            start_send(d)

        zx = jnp.zeros((1, sy, sz), jnp.float32)
        zy = jnp.zeros((sx, 1, sz), jnp.float32)
        zz = jnp.zeros((sx, sy, 1), jnp.float32)
        xm = jnp.concatenate([zx, uv[:-1, :, :]], axis=0)
        xp = jnp.concatenate([uv[1:, :, :], zx], axis=0)
        ym = jnp.concatenate([zy, uv[:, :-1, :]], axis=1)
        yp = jnp.concatenate([uv[:, 1:, :], zy], axis=1)
        zm = jnp.concatenate([zz, uv[:, :, :-1]], axis=2)
        zp = jnp.concatenate([uv[:, :, 1:], zz], axis=2)
        v = xm + xp + ym + yp + zm + zp - 6.0 * uv

        ii = lax.broadcasted_iota(jnp.int32, (sx, sy, sz), 0)
        jj = lax.broadcasted_iota(jnp.int32, (sx, sy, sz), 1)
        kk = lax.broadcasted_iota(jnp.int32, (sx, sy, sz), 2)
        gi = my_x * sx + ii
        gj = my_y * sy + jj
        gk = my_z * sz + kk
        interior = (
            (gi > 0) & (gi < NX * sx - 1)
            & (gj > 0) & (gj < NY * sy - 1)
            & (gk > 0) & (gk < NZ * sz - 1)
        )
        out_ref[...] = jnp.where(interior, v, 0.0)

        j2 = lax.broadcasted_iota(jnp.int32, (sy, sz), 0)
        k2 = lax.broadcasted_iota(jnp.int32, (sy, sz), 1)
        i2 = lax.broadcasted_iota(jnp.int32, (sx, sz), 0)
        k3 = lax.broadcasted_iota(jnp.int32, (sx, sz), 1)
        i4 = lax.broadcasted_iota(jnp.int32, (sx, sy), 0)
        j4 = lax.broadcasted_iota(jnp.int32, (sx, sy), 1)
        gj_ok = lambda a: (my_y * sy + a > 0) & (my_y * sy + a < NY * sy - 1)
        gk_ok = lambda a: (my_z * sz + a > 0) & (my_z * sz + a < NZ * sz - 1)
        gi_ok = lambda a: (my_x * sx + a > 0) & (my_x * sx + a < NX * sx - 1)
        m_yz = gj_ok(j2) & gk_ok(k2)
        m_xz = gi_ok(i2) & gk_ok(k3)
        m_xy = gi_ok(i4) & gj_ok(j4)

        face_idx = [
            (0, None, None), (sx - 1, None, None),
            (None, 0, None), (None, sy - 1, None),
            (None, None, 0), (None, None, sz - 1),
        ]
        face_mask = [m_yz, m_yz, m_xz, m_xz, m_xy, m_xy]

        def wait_and_add(d):
            @pl.when(preds[d])
            def _():
                w = pltpu.make_async_remote_copy(
                    src_ref=send_buf.at[d],
                    dst_ref=recv_buf.at[d],
                    send_sem=send_sems.at[d],
                    recv_sem=recv_sems.at[d],
                    device_id=nbrs[d],
                    device_id_type=_DEVICE_ID_TYPE.MESH,
                )
                w.wait_recv()
                halo = jnp.where(face_mask[d], recv_buf[d, :, :], 0.0)
                fi, fj, fk = face_idx[d]
                if fi is not None:
                    out_ref[fi, :, :] = out_ref[fi, :, :] + halo
                elif fj is not None:
                    out_ref[:, fj, :] = out_ref[:, fj, :] + halo
                else:
                    out_ref[:, :, fk] = out_ref[:, :, fk] + halo
                w.wait_send()

        for d in range(6):
            wait_and_add(d)

    return pl.pallas_call(
        body,
        out_shape=jax.ShapeDtypeStruct((sx, sy, sz), jnp.float32),
        in_specs=[pl.BlockSpec(memory_space=pltpu.VMEM)],
        out_specs=pl.BlockSpec(memory_space=pltpu.VMEM),
        scratch_shapes=[
            pltpu.VMEM((6, sy, sz), jnp.float32),
            pltpu.VMEM((6, sy, sz), jnp.float32),
            pltpu.SemaphoreType.DMA((6,)),
            pltpu.SemaphoreType.DMA((6,)),
        ],
        compiler_params=pltpu.CompilerParams(collective_id=0),
    )(u)


# device time: 6330 ns/iter; 2.1716x vs baseline; 1.2027x over previous
import jax
import jax.numpy as jnp
from jax import lax
from jax.experimental import pallas as pl
from jax.experimental.pallas import tpu as pltpu

NX, NY, NZ = 2, 2, 4

_DEVICE_ID_TYPE = getattr(pl, "DeviceIdType", None) or pltpu.DeviceIdType


def kernel(u):
    sx, sy, sz = u.shape

    def body(x_ref, out_ref, send_buf, recv_buf, send_sems, recv_sems):
        my_x = lax.axis_index("x")
        my_y = lax.axis_index("y")
        my_z = lax.axis_index("z")

        preds = [
            my_x > 0, my_x < NX - 1,
            my_y > 0, my_y < NY - 1,
            my_z > 0, my_z < NZ - 1,
        ]
        nbrs = [
            (my_x - 1, my_y, my_z), (my_x + 1, my_y, my_z),
            (my_x, my_y - 1, my_z), (my_x, my_y + 1, my_z),
            (my_x, my_y, my_z - 1), (my_x, my_y, my_z + 1),
        ]
        opp = [1, 0, 3, 2, 5, 4]
        send_buf[0, :, :] = x_ref[:, :, 0]
        send_buf[1, :, :] = x_ref[:, :, sz - 1]
        srcs = [
            x_ref.at[0], x_ref.at[sx - 1],
            x_ref.at[:, 0, :], x_ref.at[:, sy - 1, :],
            send_buf.at[0], send_buf.at[1],
        ]

        barrier_sem = pltpu.get_barrier_semaphore()
        for d in range(6):
            @pl.when(preds[d])
            def _(d=d):
                pl.semaphore_signal(
                    barrier_sem, inc=1,
                    device_id=nbrs[d],
                    device_id_type=_DEVICE_ID_TYPE.MESH,
                )

            @pl.when(jnp.logical_not(preds[d]))
            def _():
                pl.semaphore_signal(barrier_sem, inc=1)
        pl.semaphore_wait(barrier_sem, 6)

        def start_send(d):
            @pl.when(preds[d])
            def _():
                rdma = pltpu.make_async_remote_copy(
                    src_ref=srcs[d],
                    dst_ref=recv_buf.at[opp[d]],
                    send_sem=send_sems.at[d],
                    recv_sem=recv_sems.at[opp[d]],
                    device_id=nbrs[d],
                    device_id_type=_DEVICE_ID_TYPE.MESH,
                )
                rdma.start()

        for d in range(6):
            start_send(d)

        uv = x_ref[...]
        zx = jnp.zeros((1, sy, sz), jnp.float32)
        zy = jnp.zeros((sx, 1, sz), jnp.float32)
        zz = jnp.zeros((sx, sy, 1), jnp.float32)
        xm = jnp.concatenate([zx, uv[:-1, :, :]], axis=0)
        xp = jnp.concatenate([uv[1:, :, :], zx], axis=0)
        ym = jnp.concatenate([zy, uv[:, :-1, :]], axis=1)
        yp = jnp.concatenate([uv[:, 1:, :], zy], axis=1)
        zm = jnp.concatenate([zz, uv[:, :, :-1]], axis=2)
        zp = jnp.concatenate([uv[:, :, 1:], zz], axis=2)
        out_ref[...] = xm + xp + ym + yp + zm + zp - 6.0 * uv

        face_idx = [
            (0, None, None), (sx - 1, None, None),
            (None, 0, None), (None, sy - 1, None),
            (None, None, 0), (None, None, sz - 1),
        ]

        def face_set(d, update):
            fi, fj, fk = face_idx[d]
            if fi is not None:
                out_ref[fi, :, :] = update(out_ref[fi, :, :])
            elif fj is not None:
                out_ref[:, fj, :] = update(out_ref[:, fj, :])
            else:
                out_ref[:, :, fk] = update(out_ref[:, :, fk])

        def wait_and_add(d):
            @pl.when(preds[d])
            def _():
                w = pltpu.make_async_remote_copy(
                    src_ref=srcs[d],
                    dst_ref=recv_buf.at[d],
                    send_sem=send_sems.at[d],
                    recv_sem=recv_sems.at[d],
                    device_id=nbrs[d],
                    device_id_type=_DEVICE_ID_TYPE.MESH,
                )
                w.wait_recv()
                face_set(d, lambda cur: cur + recv_buf[d, :, :])
                w.wait_send()

        for d in range(6):
            wait_and_add(d)

        for d in range(6):
            @pl.when(jnp.logical_not(preds[d]))
            def _(d=d):
                face_set(d, lambda cur: jnp.zeros_like(cur))

    return pl.pallas_call(
        body,
        out_shape=jax.ShapeDtypeStruct((sx, sy, sz), jnp.float32),
        in_specs=[pl.BlockSpec(memory_space=pltpu.VMEM)],
        out_specs=pl.BlockSpec(memory_space=pltpu.VMEM),
        scratch_shapes=[
            pltpu.VMEM((2, sx, sy), jnp.float32),
            pltpu.VMEM((6, sy, sz), jnp.float32),
            pltpu.SemaphoreType.DMA((6,)),
            pltpu.SemaphoreType.DMA((6,)),
        ],
        compiler_params=pltpu.CompilerParams(collective_id=0),
    )(u)
